# baseline (device time: 25774 ns/iter reference)
import jax
import jax.numpy as jnp
from jax import lax
from jax.experimental import pallas as pl
from jax.experimental.pallas import tpu as pltpu

M = 1024
N = 1024
R = 4
RB = M // R


def kernel(x, w_mat):
    def body(x_ref, w_ref, out_ref, xv, wv, acc, r0, r1,
             ssem, rsem, isem, osem):
        p = lax.axis_index("i")
        b0 = p & 1
        b1 = p >> 1
        n1 = p ^ 1
        n2 = 3 - p

        xcp = pltpu.make_async_copy(x_ref, xv, isem.at[0])
        wcp = pltpu.make_async_copy(w_ref, wv, isem.at[1])
        xcp.start()
        wcp.start()
        barrier = pltpu.get_barrier_semaphore()
        pl.semaphore_signal(barrier, inc=1, device_id=(n1,),
                            device_id_type=pl.DeviceIdType.MESH)
        pl.semaphore_signal(barrier, inc=1, device_id=(n2,),
                            device_id_type=pl.DeviceIdType.MESH)
        pl.semaphore_wait(barrier, 2)
        xcp.wait()
        wcp.wait()

        k1 = 256 * (((p + 1) >> 1) & 1)
        k2 = 512 + 256 * b1
        o1 = k1 + 128 * b1
        o2 = k2 + 128 * b0

        WIDTH = [256, 128, 128, 256]
        SENDS = [
            [(n1, 256 - k1), (n2, 768 - 256 * b1)],
            [(n2, k1 + 128 - 128 * b1),
             (n1, k2 + 128 - 128 * b0)],
            [(n2, o1), (n1, o2)],
            [(n1, k1), (n2, k2)],
        ]
        RBUF = {0: r0, 1: r1}
        ACC = [[(k1, 256), (k2, 256)], [(o1, 128), (o2, 128)]]

        def start_exchange(phase, blk):
            width = WIDTH[phase]
            rows = pl.ds(blk * RB, RB)
            rdmas = []
            for j, (partner, col) in enumerate(SENDS[phase]):
                src = acc.at[rows, pl.ds(col, width)]
                if phase <= 1:
                    dst = RBUF[phase].at[blk, j]
                else:
                    dst = acc.at[rows, pl.ds(col, width)]
                rdma = pltpu.make_async_remote_copy(
                    src_ref=src,
                    dst_ref=dst,
                    send_sem=ssem.at[phase * 2 * R + blk * 2 + j],
                    recv_sem=rsem.at[phase * 2 * R + blk * 2 + j],
                    device_id=(partner,),
                    device_id_type=pl.DeviceIdType.MESH,
                )
                rdma.start()
                rdmas.append(rdma)
            return rdmas

        def consume(phase, blk):
            for rd in pending[blk]:
                rd.wait_recv()
            if phase <= 1:
                rows = pl.ds(blk * RB, RB)
                for j, (col, w) in enumerate(ACC[phase]):
                    sl = (rows, pl.ds(col, w))
                    acc[sl] = acc[sl] + RBUF[phase][blk, j, :, :]

        wb = wv[...]
        done = []

        pending = []
        for blk in range(R):
            rows = pl.ds(blk * RB, RB)
            acc[rows, :] = jnp.dot(
                xv[rows, :], wb,
                preferred_element_type=jnp.float32).astype(jnp.bfloat16)
            pending.append(start_exchange(0, blk))

        for phase in (1, 2, 3):
            nxt = []
            for blk in range(R):
                consume(phase - 1, blk)
                done.extend(pending[blk])
                nxt.append(start_exchange(phase, blk))
            pending = nxt

        ocps = []
        for blk in range(R):
            consume(3, blk)
            done.extend(pending[blk])
            rows = pl.ds(blk * RB, RB)
            ocp = pltpu.make_async_copy(
                acc.at[rows, :], out_ref.at[rows, :], osem.at[blk])
            ocp.start()
            ocps.append(ocp)

        for ocp in ocps:
            ocp.wait()
        for rd in done:
            rd.wait_send()

    return pl.pallas_call(
        body,
        out_shape=jax.ShapeDtypeStruct((M, N), jnp.bfloat16),
        in_specs=[
            pl.BlockSpec(memory_space=pl.ANY),
            pl.BlockSpec(memory_space=pl.ANY),
        ],
        out_specs=pl.BlockSpec(memory_space=pl.ANY),
        scratch_shapes=[
            pltpu.VMEM((M, 256), jnp.bfloat16),
            pltpu.VMEM((256, N), jnp.bfloat16),
            pltpu.VMEM((M, N), jnp.bfloat16),
            pltpu.VMEM((R, 2, RB, 256), jnp.bfloat16),
            pltpu.VMEM((R, 2, RB, 128), jnp.bfloat16),
            pltpu.SemaphoreType.DMA((4 * 2 * R,)),
            pltpu.SemaphoreType.DMA((4 * 2 * R,)),
            pltpu.SemaphoreType.DMA((2,)),
            pltpu.SemaphoreType.DMA((R,)),
        ],
        compiler_params=pltpu.CompilerParams(collective_id=0),
    )(x.astype(jnp.bfloat16), w_mat.astype(jnp.bfloat16))


# device time: 24676 ns/iter; 1.0445x vs baseline; 1.0445x over previous
import jax
import jax.numpy as jnp
from jax import lax
from jax.experimental import pallas as pl
from jax.experimental.pallas import tpu as pltpu

M = 1024
N = 1024
R = 4
RB = M // R


def kernel(x, w_mat):
    def body(x_ref, w_ref, out_ref, r0, r1, ssem, rsem):
        p = lax.axis_index("i")
        b0 = p & 1
        b1 = p >> 1
        n1 = p ^ 1
        n2 = 3 - p

        k1 = 256 * (((p + 1) >> 1) & 1)
        k2 = 512 + 256 * b1
        o1 = k1 + 128 * b1
        o2 = k2 + 128 * b0

        WIDTH = [256, 128, 128, 256]
        SENDS = [
            [(n1, 256 - k1), (n2, 768 - 256 * b1)],
            [(n2, k1 + 128 - 128 * b1),
             (n1, k2 + 128 - 128 * b0)],
            [(n2, o1), (n1, o2)],
            [(n1, k1), (n2, k2)],
        ]
        RBUF = {0: r0, 1: r1}
        ACC = [[(k1, 256), (k2, 256)], [(o1, 128), (o2, 128)]]

        def start_exchange(phase, blk):
            width = WIDTH[phase]
            rows = pl.ds(blk * RB, RB)
            rdmas = []
            for j, (partner, col) in enumerate(SENDS[phase]):
                src = out_ref.at[rows, pl.ds(col, width)]
                if phase <= 1:
                    dst = RBUF[phase].at[blk, j]
                else:
                    dst = out_ref.at[rows, pl.ds(col, width)]
                rdma = pltpu.make_async_remote_copy(
                    src_ref=src,
                    dst_ref=dst,
                    send_sem=ssem.at[phase * 2 * R + blk * 2 + j],
                    recv_sem=rsem.at[phase * 2 * R + blk * 2 + j],
                    device_id=(partner,),
                    device_id_type=pl.DeviceIdType.MESH,
                )
                rdma.start()
                rdmas.append(rdma)
            return rdmas

        def consume(phase, blk):
            for rd in pending[blk]:
                rd.wait_recv()
            if phase <= 1:
                rows = pl.ds(blk * RB, RB)
                for j, (col, w) in enumerate(ACC[phase]):
                    sl = (rows, pl.ds(col, w))
                    out_ref[sl] = out_ref[sl] + RBUF[phase][blk, j, :, :]

        wb = w_ref[...]
        done = []

        pending = []
        for blk in range(R):
            rows = pl.ds(blk * RB, RB)
            out_ref[rows, :] = jnp.dot(
                x_ref[rows, :], wb,
                preferred_element_type=jnp.float32).astype(jnp.bfloat16)
            if blk == 0:
                barrier = pltpu.get_barrier_semaphore()
                pl.semaphore_signal(barrier, inc=1, device_id=(n1,),
                                    device_id_type=pl.DeviceIdType.MESH)
                pl.semaphore_signal(barrier, inc=1, device_id=(n2,),
                                    device_id_type=pl.DeviceIdType.MESH)
                pl.semaphore_wait(barrier, 2)
            pending.append(start_exchange(0, blk))

        for phase in (1, 2, 3):
            nxt = []
            for blk in range(R):
                consume(phase - 1, blk)
                done.extend(pending[blk])
                nxt.append(start_exchange(phase, blk))
            pending = nxt
        for blk in range(R):
            consume(3, blk)
            done.extend(pending[blk])

        for rd in done:
            rd.wait_send()

    return pl.pallas_call(
        body,
        out_shape=jax.ShapeDtypeStruct((M, N), jnp.bfloat16),
        in_specs=[
            pl.BlockSpec(memory_space=pltpu.VMEM),
            pl.BlockSpec(memory_space=pltpu.VMEM),
        ],
        out_specs=pl.BlockSpec(memory_space=pltpu.VMEM),
        scratch_shapes=[
            pltpu.VMEM((R, 2, RB, 256), jnp.bfloat16),
            pltpu.VMEM((R, 2, RB, 128), jnp.bfloat16),
            pltpu.SemaphoreType.DMA((4 * 2 * R,)),
            pltpu.SemaphoreType.DMA((4 * 2 * R,)),
        ],
        compiler_params=pltpu.CompilerParams(collective_id=0),
    )(x.astype(jnp.bfloat16), w_mat.astype(jnp.bfloat16))
